# Initial kernel scaffold; baseline (speedup 1.0000x reference)
import jax
import jax.numpy as jnp
from jax.experimental import pallas as pl


def kernel(item_hist, cate_hist, price_hist, emb_item, emb_cate, W_price, bn_gamma, bn_beta, bn_mean, bn_var):
    raise NotImplementedError("write your pallas kernel here")


# speedup vs baseline: 1.2666x; 1.2666x over previous
import jax
import jax.numpy as jnp
from jax import lax
from jax.experimental import pallas as pl
from jax.experimental.pallas import tpu as pltpu
from jax.experimental.pallas import tpu_sc as plsc

B = 4096
L = 200
N = B * L
EMB = 32
CU = 16
OUT_W = 2 * EMB + CU
NW = 32
PER_W = N // NW
CHUNK = 256
N_CH = PER_W // CHUNK
IDX_ROWS = CHUNK // 128


def _sc_body(qi_i, lb_i, qi_c, lb_c, price_h, embi4, embc4, wg_h, wb_h, out,
             qiv_i, lbv_i, qiv_c, lbv_c, pv, ri_v, rc_v, asm_v,
             wg_v, wb_v, gsem):
    c = lax.axis_index("c")
    s = lax.axis_index("s")
    wid = s * 2 + c

    pltpu.sync_copy(wg_h, wg_v)
    pltpu.sync_copy(wb_h, wb_v)
    wg = wg_v[...]
    wb = wb_v[...]
    lane = lax.iota(jnp.int32, 16)

    def chunk_body(ch, carry):
        base = wid * PER_W + ch * CHUNK

        pltpu.sync_copy(qi_i.at[pl.ds(base, CHUNK)], qiv_i)
        pltpu.sync_copy(qi_c.at[pl.ds(base, CHUNK)], qiv_c)
        pltpu.sync_copy(lb_i.at[pl.ds(base, CHUNK)], lbv_i)
        pltpu.sync_copy(lb_c.at[pl.ds(base, CHUNK)], lbv_c)
        pltpu.sync_copy(price_h.at[pl.ds(base, CHUNK)], pv)

        descs = []
        for j in range(IDX_ROWS):
            descs.append(pltpu.async_copy(
                embi4.at[qiv_i.at[pl.ds(j * 128, 128)]],
                ri_v.at[pl.ds(j * 128, 128)], gsem))
            descs.append(pltpu.async_copy(
                embc4.at[qiv_c.at[pl.ds(j * 128, 128)]],
                rc_v.at[pl.ds(j * 128, 128)], gsem))
        for d in descs:
            d.wait()

        def prow(n, cc):
            row = jnp.full((16,), n, jnp.int32)
            obase = n * OUT_W
            li = plsc.load_gather(lbv_i, [row]) + lane
            a0 = plsc.load_gather(ri_v, [row, li])
            a1 = plsc.load_gather(ri_v, [row, li + 16])
            asm_v[pl.ds(obase, 16)] = a0
            asm_v[pl.ds(obase + 16, 16)] = a1
            lc = plsc.load_gather(lbv_c, [row]) + lane
            b0 = plsc.load_gather(rc_v, [row, lc])
            b1 = plsc.load_gather(rc_v, [row, lc + 16])
            asm_v[pl.ds(obase + 32, 16)] = b0
            asm_v[pl.ds(obase + 48, 16)] = b1
            p = plsc.load_gather(pv, [row])
            asm_v[pl.ds(obase + 64, 16)] = p * wg + wb
            return cc

        lax.fori_loop(0, CHUNK, prow, 0, unroll=2)

        pltpu.sync_copy(asm_v, out.at[pl.ds(base * OUT_W, CHUNK * OUT_W)])
        return carry

    lax.fori_loop(0, N_CH, chunk_body, 0)


@jax.jit
def _run(qi_i, lb_i, qi_c, lb_c, price_flat, embi4, embc4, wg, wb):
    mesh = plsc.VectorSubcoreMesh(core_axis_name="c", subcore_axis_name="s")
    return pl.kernel(
        _sc_body,
        out_type=jax.ShapeDtypeStruct((N * OUT_W,), jnp.float32),
        mesh=mesh,
        compiler_params=pltpu.CompilerParams(needs_layout_passes=False),
        scratch_types=[
            pltpu.VMEM((CHUNK,), jnp.int32),
            pltpu.VMEM((CHUNK,), jnp.int32),
            pltpu.VMEM((CHUNK,), jnp.int32),
            pltpu.VMEM((CHUNK,), jnp.int32),
            pltpu.VMEM((CHUNK,), jnp.float32),
            pltpu.VMEM((CHUNK, 128), jnp.float32),
            pltpu.VMEM((CHUNK, 128), jnp.float32),
            pltpu.VMEM((CHUNK * OUT_W,), jnp.float32),
            pltpu.VMEM((CU,), jnp.float32),
            pltpu.VMEM((CU,), jnp.float32),
            pltpu.SemaphoreType.DMA,
        ],
    )(qi_i, lb_i, qi_c, lb_c, price_flat, embi4, embc4, wg, wb)


def kernel(item_hist, cate_hist, price_hist, emb_item, emb_cate, W_price,
           bn_gamma, bn_beta, bn_mean, bn_var):
    g = bn_gamma / jnp.sqrt(bn_var + 1e-3)
    wg = (W_price[0] * g).astype(jnp.float32)
    wb = (W_price[0] * (bn_beta - bn_mean * g)).astype(jnp.float32)

    item_flat = item_hist.reshape(N)
    cate_flat = cate_hist.reshape(N)
    qi_i = item_flat >> 2
    lb_i = (item_flat & 3) << 5
    qi_c = cate_flat >> 2
    lb_c = (cate_flat & 3) << 5
    price_flat = price_hist.reshape(N)

    vi = emb_item.shape[0]
    vc = emb_cate.shape[0]
    embi4 = jnp.pad(emb_item, ((0, (-vi) % 4), (0, 0))).reshape(-1, 128)
    embc4 = jnp.pad(emb_cate, ((0, (-vc) % 4), (0, 0))).reshape(-1, 128)

    out = _run(qi_i, lb_i, qi_c, lb_c, price_flat, embi4, embc4, wg, wb)
    return out.reshape(B, L, OUT_W)


# speedup vs baseline: 1.4239x; 1.1242x over previous
import jax
import jax.numpy as jnp
from jax import lax
from jax.experimental import pallas as pl
from jax.experimental.pallas import tpu as pltpu
from jax.experimental.pallas import tpu_sc as plsc

B = 4096
L = 200
N = B * L
EMB = 32
CU = 16
OUT_W = 2 * EMB + CU
NW = 32
PER_W = N // NW
CHUNK = 256
N_CH = PER_W // CHUNK
IDX_ROWS = CHUNK // 128


def _sc_body(qi_i, lb_i, qi_c, lb_c, price_h, embi4, embc4, wg_h, wb_h, out,
             qiv_i, lbv_i, qiv_c, lbv_c, pv, ri_v, rc_v, asm_v,
             wg_v, wb_v, gsem):
    c = lax.axis_index("c")
    s = lax.axis_index("s")
    wid = s * 2 + c

    pltpu.sync_copy(wg_h, wg_v)
    pltpu.sync_copy(wb_h, wb_v)
    wg = wg_v[...]
    wb = wb_v[...]
    lane = lax.iota(jnp.int32, 16)

    def chunk_body(ch, carry):
        base = wid * PER_W + ch * CHUNK

        pltpu.sync_copy(qi_i.at[pl.ds(base, CHUNK)], qiv_i)
        pltpu.sync_copy(qi_c.at[pl.ds(base, CHUNK)], qiv_c)
        pltpu.sync_copy(lb_i.at[pl.ds(base, CHUNK)], lbv_i)
        pltpu.sync_copy(lb_c.at[pl.ds(base, CHUNK)], lbv_c)
        pltpu.sync_copy(price_h.at[pl.ds(base, CHUNK)], pv)

        descs = []
        for j in range(IDX_ROWS):
            descs.append(pltpu.async_copy(
                embi4.at[qiv_i.at[pl.ds(j * 128, 128)]],
                ri_v.at[pl.ds(j * 128, 128)], gsem))
            descs.append(pltpu.async_copy(
                embc4.at[qiv_c.at[pl.ds(j * 128, 128)]],
                rc_v.at[pl.ds(j * 128, 128)], gsem))
        for d in descs:
            d.wait()

        def prow(n, cc):
            row = jnp.full((16,), n, jnp.int32)
            li = plsc.load_gather(lbv_i, [row]) + lane
            a0 = plsc.load_gather(ri_v, [row, li])
            a1 = plsc.load_gather(ri_v, [row, li + 16])
            asm_v[n, pl.ds(0, 16)] = a0
            asm_v[n, pl.ds(16, 16)] = a1
            lc = plsc.load_gather(lbv_c, [row]) + lane
            b0 = plsc.load_gather(rc_v, [row, lc])
            b1 = plsc.load_gather(rc_v, [row, lc + 16])
            asm_v[n, pl.ds(32, 16)] = b0
            asm_v[n, pl.ds(48, 16)] = b1
            p = plsc.load_gather(pv, [row])
            asm_v[n, pl.ds(64, 16)] = p * wg + wb
            return cc

        lax.fori_loop(0, CHUNK, prow, 0, unroll=2)

        pltpu.sync_copy(asm_v, out.at[pl.ds(base, CHUNK)])
        return carry

    lax.fori_loop(0, N_CH, chunk_body, 0)


@jax.jit
def _run(qi_i, lb_i, qi_c, lb_c, price_flat, embi4, embc4, wg, wb):
    mesh = plsc.VectorSubcoreMesh(core_axis_name="c", subcore_axis_name="s")
    return pl.kernel(
        _sc_body,
        out_type=jax.ShapeDtypeStruct((N, OUT_W), jnp.float32),
        mesh=mesh,
        compiler_params=pltpu.CompilerParams(needs_layout_passes=False),
        scratch_types=[
            pltpu.VMEM((CHUNK,), jnp.int32),
            pltpu.VMEM((CHUNK,), jnp.int32),
            pltpu.VMEM((CHUNK,), jnp.int32),
            pltpu.VMEM((CHUNK,), jnp.int32),
            pltpu.VMEM((CHUNK,), jnp.float32),
            pltpu.VMEM((CHUNK, 128), jnp.float32),
            pltpu.VMEM((CHUNK, 128), jnp.float32),
            pltpu.VMEM((CHUNK, OUT_W), jnp.float32),
            pltpu.VMEM((CU,), jnp.float32),
            pltpu.VMEM((CU,), jnp.float32),
            pltpu.SemaphoreType.DMA,
        ],
    )(qi_i, lb_i, qi_c, lb_c, price_flat, embi4, embc4, wg, wb)


def kernel(item_hist, cate_hist, price_hist, emb_item, emb_cate, W_price,
           bn_gamma, bn_beta, bn_mean, bn_var):
    g = bn_gamma / jnp.sqrt(bn_var + 1e-3)
    wg = (W_price[0] * g).astype(jnp.float32)
    wb = (W_price[0] * (bn_beta - bn_mean * g)).astype(jnp.float32)

    item_flat = item_hist.reshape(N)
    cate_flat = cate_hist.reshape(N)
    qi_i = item_flat >> 2
    lb_i = (item_flat & 3) << 5
    qi_c = cate_flat >> 2
    lb_c = (cate_flat & 3) << 5
    price_flat = price_hist.reshape(N)

    vi = emb_item.shape[0]
    vc = emb_cate.shape[0]
    embi4 = jnp.pad(emb_item, ((0, (-vi) % 4), (0, 0))).reshape(-1, 128)
    embc4 = jnp.pad(emb_cate, ((0, (-vc) % 4), (0, 0))).reshape(-1, 128)

    out = _run(qi_i, lb_i, qi_c, lb_c, price_flat, embi4, embc4, wg, wb)
    return out.reshape(B, L, OUT_W)


# speedup vs baseline: 1.5346x; 1.0778x over previous
import jax
import jax.numpy as jnp
from jax import lax
from jax.experimental import pallas as pl
from jax.experimental.pallas import tpu as pltpu
from jax.experimental.pallas import tpu_sc as plsc

B = 4096
L = 200
N = B * L
EMB = 32
CU = 16
OUT_W = 2 * EMB + CU
NW = 32
PER_W = N // NW
CHUNK = 128
N_CH = PER_W // CHUNK
T = N_CH // 2


def _sc_body(qi_i, lb_i, qi_c, lb_c, price_h, embi4, embc4, wg_h, wb_h, out,
             qiv_i0, lbv_i0, qiv_c0, lbv_c0, pv0, ri0, rc0, asm0,
             qiv_i1, lbv_i1, qiv_c1, lbv_c1, pv1, ri1, rc1, asm1,
             wg_v, wb_v, gsem0, gsem1, wsem0, wsem1):
    c = lax.axis_index("c")
    s = lax.axis_index("s")
    wid = s * 2 + c

    pltpu.sync_copy(wg_h, wg_v)
    pltpu.sync_copy(wb_h, wb_v)
    wg = wg_v[...]
    wb = wb_v[...]
    lane = lax.iota(jnp.int32, 16)

    slot0 = (qiv_i0, lbv_i0, qiv_c0, lbv_c0, pv0, ri0, rc0, asm0,
             gsem0, wsem0)
    slot1 = (qiv_i1, lbv_i1, qiv_c1, lbv_c1, pv1, ri1, rc1, asm1,
             gsem1, wsem1)

    def issue(ch, sl):
        (qiv_i, lbv_i, qiv_c, lbv_c, pv, ri, rc, asm, gsem, wsem) = sl
        base = wid * PER_W + ch * CHUNK
        pltpu.sync_copy(qi_i.at[pl.ds(base, CHUNK)], qiv_i)
        pltpu.sync_copy(qi_c.at[pl.ds(base, CHUNK)], qiv_c)
        pltpu.sync_copy(lb_i.at[pl.ds(base, CHUNK)], lbv_i)
        pltpu.sync_copy(lb_c.at[pl.ds(base, CHUNK)], lbv_c)
        pltpu.sync_copy(price_h.at[pl.ds(base, CHUNK)], pv)
        pltpu.async_copy(embi4.at[qiv_i], ri, gsem)
        pltpu.async_copy(embc4.at[qiv_c], rc, gsem)

    def process(ch, sl, first):
        (qiv_i, lbv_i, qiv_c, lbv_c, pv, ri, rc, asm, gsem, wsem) = sl
        base = wid * PER_W + ch * CHUNK
        pltpu.make_async_copy(embi4.at[pl.ds(0, CHUNK)], ri, gsem).wait()
        pltpu.make_async_copy(embc4.at[pl.ds(0, CHUNK)], rc, gsem).wait()

        @pl.when(jnp.logical_not(first))
        def _():
            pltpu.make_async_copy(asm, out.at[pl.ds(base, CHUNK)],
                                  wsem).wait()

        def prow(n, cc):
            row = jnp.full((16,), n, jnp.int32)
            li = plsc.load_gather(lbv_i, [row]) + lane
            a0 = plsc.load_gather(ri, [row, li])
            a1 = plsc.load_gather(ri, [row, li + 16])
            asm[n, pl.ds(0, 16)] = a0
            asm[n, pl.ds(16, 16)] = a1
            lc = plsc.load_gather(lbv_c, [row]) + lane
            b0 = plsc.load_gather(rc, [row, lc])
            b1 = plsc.load_gather(rc, [row, lc + 16])
            asm[n, pl.ds(32, 16)] = b0
            asm[n, pl.ds(48, 16)] = b1
            p = plsc.load_gather(pv, [row])
            asm[n, pl.ds(64, 16)] = p * wg + wb
            return cc

        lax.fori_loop(0, CHUNK, prow, 0, unroll=2)

        pltpu.async_copy(asm, out.at[pl.ds(base, CHUNK)], wsem)

    issue(0, slot0)

    def pair_body(t, carry):
        issue(2 * t + 1, slot1)
        process(2 * t, slot0, t == 0)

        @pl.when(t < T - 1)
        def _():
            issue(2 * t + 2, slot0)

        process(2 * t + 1, slot1, t == 0)
        return carry

    lax.fori_loop(0, T, pair_body, 0)

    pltpu.make_async_copy(asm0, out.at[pl.ds(0, CHUNK)], wsem0).wait()
    pltpu.make_async_copy(asm1, out.at[pl.ds(0, CHUNK)], wsem1).wait()


@jax.jit
def _run(qi_i, lb_i, qi_c, lb_c, price_flat, embi4, embc4, wg, wb):
    mesh = plsc.VectorSubcoreMesh(core_axis_name="c", subcore_axis_name="s")
    slot_scratch = [
        pltpu.VMEM((CHUNK,), jnp.int32),
        pltpu.VMEM((CHUNK,), jnp.int32),
        pltpu.VMEM((CHUNK,), jnp.int32),
        pltpu.VMEM((CHUNK,), jnp.int32),
        pltpu.VMEM((CHUNK,), jnp.float32),
        pltpu.VMEM((CHUNK, 128), jnp.float32),
        pltpu.VMEM((CHUNK, 128), jnp.float32),
        pltpu.VMEM((CHUNK, OUT_W), jnp.float32),
    ]
    return pl.kernel(
        _sc_body,
        out_type=jax.ShapeDtypeStruct((N, OUT_W), jnp.float32),
        mesh=mesh,
        compiler_params=pltpu.CompilerParams(needs_layout_passes=False),
        scratch_types=slot_scratch + slot_scratch + [
            pltpu.VMEM((CU,), jnp.float32),
            pltpu.VMEM((CU,), jnp.float32),
            pltpu.SemaphoreType.DMA,
            pltpu.SemaphoreType.DMA,
            pltpu.SemaphoreType.DMA,
            pltpu.SemaphoreType.DMA,
        ],
    )(qi_i, lb_i, qi_c, lb_c, price_flat, embi4, embc4, wg, wb)


def kernel(item_hist, cate_hist, price_hist, emb_item, emb_cate, W_price,
           bn_gamma, bn_beta, bn_mean, bn_var):
    g = bn_gamma / jnp.sqrt(bn_var + 1e-3)
    wg = (W_price[0] * g).astype(jnp.float32)
    wb = (W_price[0] * (bn_beta - bn_mean * g)).astype(jnp.float32)

    item_flat = item_hist.reshape(N)
    cate_flat = cate_hist.reshape(N)
    qi_i = item_flat >> 2
    lb_i = (item_flat & 3) << 5
    qi_c = cate_flat >> 2
    lb_c = (cate_flat & 3) << 5
    price_flat = price_hist.reshape(N)

    vi = emb_item.shape[0]
    vc = emb_cate.shape[0]
    embi4 = jnp.pad(emb_item, ((0, (-vi) % 4), (0, 0))).reshape(-1, 128)
    embc4 = jnp.pad(emb_cate, ((0, (-vc) % 4), (0, 0))).reshape(-1, 128)

    out = _run(qi_i, lb_i, qi_c, lb_c, price_flat, embi4, embc4, wg, wb)
    return out.reshape(B, L, OUT_W)


# speedup vs baseline: 1.8577x; 1.2106x over previous
import jax
import jax.numpy as jnp
from jax import lax
from jax.experimental import pallas as pl
from jax.experimental.pallas import tpu as pltpu
from jax.experimental.pallas import tpu_sc as plsc

B = 4096
L = 200
N = B * L
EMB = 32
CU = 16
OUT_W = 2 * EMB + CU
NW = 32
PER_W = N // NW
CHUNK = 128
N_CH = PER_W // CHUNK
T = N_CH // 4
PACK_W = 5 * CHUNK


def _sc_body(pack_h, embi4, embc4, wg_h, wb_h, out,
             i0, i1, i2, i3, ri0, rc0, asm0, ri1, rc1, asm1,
             wg_v, wb_v, is0, is1, is2, is3, gs0, gs1, ws0, ws1):
    c = lax.axis_index("c")
    s = lax.axis_index("s")
    wid = s * 2 + c
    cq0 = wid * N_CH

    pltpu.sync_copy(wg_h, wg_v)
    pltpu.sync_copy(wb_h, wb_v)
    wg = wg_v[...]
    wb = wb_v[...]
    lane = lax.iota(jnp.int32, 16)

    islots = (i0, i1, i2, i3)
    isems = (is0, is1, is2, is3)
    rslots = ((ri0, rc0, asm0, gs0, ws0), (ri1, rc1, asm1, gs1, ws1))

    def fetch_idx(cq, j):
        pltpu.async_copy(pack_h.at[pl.ds(cq * PACK_W, PACK_W)],
                         islots[j], isems[j])

    def fire(j, r):
        (ri, rc, asm, gsem, wsem) = rslots[r]
        I = islots[j]
        pltpu.make_async_copy(pack_h.at[pl.ds(0, PACK_W)], I,
                              isems[j]).wait()
        pltpu.async_copy(embi4.at[I.at[pl.ds(0, CHUNK)]], ri, gsem)
        pltpu.async_copy(embc4.at[I.at[pl.ds(CHUNK, CHUNK)]], rc, gsem)

    def process(cq, j, r, drain_pred):
        (ri, rc, asm, gsem, wsem) = rslots[r]
        I = islots[j]
        base = cq * CHUNK
        pltpu.make_async_copy(embi4.at[pl.ds(0, CHUNK)], ri, gsem).wait()
        pltpu.make_async_copy(embc4.at[pl.ds(0, CHUNK)], rc, gsem).wait()

        def drain_w():
            pltpu.make_async_copy(asm, out.at[pl.ds(base, CHUNK)],
                                  wsem).wait()

        if drain_pred is True:
            drain_w()
        else:
            pl.when(drain_pred)(drain_w)

        def prow(n, cc):
            row = jnp.full((16,), n, jnp.int32)
            li = plsc.load_gather(I, [row + 2 * CHUNK]) + lane
            a0 = plsc.load_gather(ri, [row, li])
            a1 = plsc.load_gather(ri, [row, li + 16])
            asm[n, pl.ds(0, 16)] = a0
            asm[n, pl.ds(16, 16)] = a1
            lc = plsc.load_gather(I, [row + 3 * CHUNK]) + lane
            b0 = plsc.load_gather(rc, [row, lc])
            b1 = plsc.load_gather(rc, [row, lc + 16])
            asm[n, pl.ds(32, 16)] = b0
            asm[n, pl.ds(48, 16)] = b1
            p = plsc.bitcast(plsc.load_gather(I, [row + 4 * CHUNK]),
                             jnp.float32)
            asm[n, pl.ds(64, 16)] = p * wg + wb
            return cc

        lax.fori_loop(0, CHUNK, prow, 0, unroll=2)
        pltpu.async_copy(asm, out.at[pl.ds(base, CHUNK)], wsem)

    pltpu.sync_copy(pack_h.at[pl.ds(cq0 * PACK_W, PACK_W)], i0)
    pltpu.sync_copy(pack_h.at[pl.ds((cq0 + 1) * PACK_W, PACK_W)], i1)
    pltpu.async_copy(embi4.at[i0.at[pl.ds(0, CHUNK)]], ri0, gs0)
    pltpu.async_copy(embc4.at[i0.at[pl.ds(CHUNK, CHUNK)]], rc0, gs0)
    pltpu.async_copy(embi4.at[i1.at[pl.ds(0, CHUNK)]], ri1, gs1)
    pltpu.async_copy(embc4.at[i1.at[pl.ds(CHUNK, CHUNK)]], rc1, gs1)
    fetch_idx(cq0 + 2, 2)
    fetch_idx(cq0 + 3, 3)

    def pair_body(t, carry):
        c0 = cq0 + 4 * t
        more = t < T - 1
        not_first = t > 0

        process(c0, 0, 0, not_first)

        @pl.when(more)
        def _():
            fetch_idx(c0 + 4, 0)
        fire(2, 0)

        process(c0 + 1, 1, 1, not_first)

        @pl.when(more)
        def _():
            fetch_idx(c0 + 5, 1)
        fire(3, 1)

        process(c0 + 2, 2, 0, True)

        @pl.when(more)
        def _():
            fetch_idx(c0 + 6, 2)
            fire(0, 0)

        process(c0 + 3, 3, 1, True)

        @pl.when(more)
        def _():
            fetch_idx(c0 + 7, 3)
            fire(1, 1)
        return carry

    lax.fori_loop(0, T, pair_body, 0)

    pltpu.make_async_copy(asm0, out.at[pl.ds(0, CHUNK)], ws0).wait()
    pltpu.make_async_copy(asm1, out.at[pl.ds(0, CHUNK)], ws1).wait()


@jax.jit
def _run(pack, embi4, embc4, wg, wb):
    mesh = plsc.VectorSubcoreMesh(core_axis_name="c", subcore_axis_name="s")
    return pl.kernel(
        _sc_body,
        out_type=jax.ShapeDtypeStruct((N, OUT_W), jnp.float32),
        mesh=mesh,
        compiler_params=pltpu.CompilerParams(needs_layout_passes=False),
        scratch_types=[
            pltpu.VMEM((PACK_W,), jnp.int32),
            pltpu.VMEM((PACK_W,), jnp.int32),
            pltpu.VMEM((PACK_W,), jnp.int32),
            pltpu.VMEM((PACK_W,), jnp.int32),
            pltpu.VMEM((CHUNK, 128), jnp.float32),
            pltpu.VMEM((CHUNK, 128), jnp.float32),
            pltpu.VMEM((CHUNK, OUT_W), jnp.float32),
            pltpu.VMEM((CHUNK, 128), jnp.float32),
            pltpu.VMEM((CHUNK, 128), jnp.float32),
            pltpu.VMEM((CHUNK, OUT_W), jnp.float32),
            pltpu.VMEM((CU,), jnp.float32),
            pltpu.VMEM((CU,), jnp.float32),
            pltpu.SemaphoreType.DMA,
            pltpu.SemaphoreType.DMA,
            pltpu.SemaphoreType.DMA,
            pltpu.SemaphoreType.DMA,
            pltpu.SemaphoreType.DMA,
            pltpu.SemaphoreType.DMA,
            pltpu.SemaphoreType.DMA,
            pltpu.SemaphoreType.DMA,
        ],
    )(pack, embi4, embc4, wg, wb)


def kernel(item_hist, cate_hist, price_hist, emb_item, emb_cate, W_price,
           bn_gamma, bn_beta, bn_mean, bn_var):
    g = bn_gamma / jnp.sqrt(bn_var + 1e-3)
    wg = (W_price[0] * g).astype(jnp.float32)
    wb = (W_price[0] * (bn_beta - bn_mean * g)).astype(jnp.float32)

    item_flat = item_hist.reshape(N)
    cate_flat = cate_hist.reshape(N)
    price_flat = price_hist.reshape(N)
    price_bits = lax.bitcast_convert_type(price_flat, jnp.int32)
    nch = N // CHUNK
    pack = jnp.stack(
        [(item_flat >> 2).reshape(nch, CHUNK),
         (cate_flat >> 2).reshape(nch, CHUNK),
         ((item_flat & 3) << 5).reshape(nch, CHUNK),
         ((cate_flat & 3) << 5).reshape(nch, CHUNK),
         price_bits.reshape(nch, CHUNK)],
        axis=1).reshape(-1)

    vi = emb_item.shape[0]
    vc = emb_cate.shape[0]
    embi4 = jnp.pad(emb_item, ((0, (-vi) % 4), (0, 0))).reshape(-1, 128)
    embc4 = jnp.pad(emb_cate, ((0, (-vc) % 4), (0, 0))).reshape(-1, 128)

    out = _run(pack, embi4, embc4, wg, wb)
    return out.reshape(B, L, OUT_W)
